# Initial kernel scaffold; baseline (speedup 1.0000x reference)
import jax
import jax.numpy as jnp
from jax.experimental import pallas as pl


def kernel(features, cls_tokens):
    raise NotImplementedError("write your pallas kernel here")


# speedup vs baseline: 47.6673x; 47.6673x over previous
import functools
import numpy as np
import jax
import jax.numpy as jnp
from jax.experimental import pallas as pl

L, B, P, D = 2, 4, 256, 1024
PH = PW = 16
H = W = 224
R_LIST = [1, 3, 5]
K_LIST = [1, 2, 3]
NR = len(R_LIST)

_INTERPRET = False


def _shift_rows(x, s):
    b, n, d = x.shape
    if s > 0:
        return jnp.concatenate([jnp.zeros((b, s, d), x.dtype), x[:, :-s, :]], axis=1)
    s = -s
    return jnp.concatenate([x[:, s:, :], jnp.zeros((b, s, d), x.dtype)], axis=1)


def _lnamd_shifts(x, r):
    if r == 1:
        return x
    p = (r - 1) // 2
    wpos = jax.lax.broadcasted_iota(jnp.int32, (1, P, 1), 1) % PW
    acc = x
    for s in range(1, p + 1):
        up = _shift_rows(x, s)
        up = jnp.where(wpos < s, 0.0, up)
        dn = _shift_rows(x, -s)
        dn = jnp.where(wpos >= PW - s, 0.0, dn)
        acc = acc + up + dn
    acc2 = acc
    for s in range(1, p + 1):
        acc2 = acc2 + _shift_rows(acc, 16 * s) + _shift_rows(acc, -16 * s)
    return acc2 / float(r * r)


def _stage_a_body(feat_ref, mins_ref, amins_ref):
    x = feat_ref[0]
    mu = jnp.mean(x, axis=(1, 2), keepdims=True)
    var = jnp.mean((x - mu) ** 2, axis=(1, 2), keepdims=True)
    x = (x - mu) / jnp.sqrt(var + 1e-5)
    for ri, r in enumerate(R_LIST):
        rf = _lnamd_shifts(x, r)
        nrm = jnp.sqrt(jnp.sum(rf * rf, axis=-1, keepdims=True))
        rf = rf / nrm
        flat = rf.reshape(B * P, D)
        g = jax.lax.dot_general(flat, flat, (((1,), (1,)), ((), ())))
        sqn = jnp.sum(flat * flat, axis=1)
        sq = (sqn[:, None] + sqn[None, :]) - 2.0 * g
        for j in range(B):
            blk = sq[:, j * P:(j + 1) * P]
            m = jnp.min(blk, axis=1)
            eq = blk == m[:, None]
            ii = jax.lax.broadcasted_iota(jnp.int32, (B * P, P), 1)
            am = jnp.min(jnp.where(eq, ii, P), axis=1)
            mins_ref[0, ri, j, :] = m
            amins_ref[0, ri, j, :] = am.astype(jnp.int32)


def _stage_c_body(mins_ref, cls_ref, ry_ref, rxt_ref, finals_ref, pix_ref):
    d = jnp.sqrt(jnp.maximum(mins_ref[...], 1e-12))
    d6 = d.reshape(L * NR, B, B * P)
    score_rows = []
    for b in range(B):
        sub = d6[:, :, b * P:(b + 1) * P]
        others = [j for j in range(B) if j != b]
        m = sub[:, others[0], :]
        for j in others[1:]:
            m = jnp.minimum(m, sub[:, j, :])
        score_rows.append(jnp.mean(m, axis=0))
    scores = jnp.stack(score_rows, axis=0)
    scores_image = jnp.max(scores, axis=1)

    cls = cls_ref[...]
    cls = cls / jnp.sqrt(jnp.sum(cls * cls, axis=1, keepdims=True))
    sim = jax.lax.dot_general(cls, cls, (((1,), (1,)), ((), ())),
                              precision=jax.lax.Precision.HIGHEST)
    col_iota = jax.lax.broadcasted_iota(jnp.int32, (B, B), 1)
    rank = jnp.zeros((B, B), jnp.int32)
    for jp in range(B):
        c = sim[:, jp:jp + 1]
        gt = (c > sim).astype(jnp.int32)
        eqb = ((c == sim) & (jp < col_iota)).astype(jnp.int32)
        rank = rank + gt + eqb
    finals = jnp.zeros((B,), jnp.float32)
    for k in K_LIST:
        mask = (rank < k).astype(jnp.float32)
        wm = sim * mask
        wm = wm / jnp.sum(wm, axis=1, keepdims=True)
        finals = finals + jnp.dot(wm, scores_image,
                                  precision=jax.lax.Precision.HIGHEST)
    finals_ref[0, :] = finals / float(len(K_LIST))

    ry = ry_ref[...]
    rxt = rxt_ref[...]
    for b in range(B):
        sp = jnp.stack([scores[b, h * PW:(h + 1) * PW] for h in range(PH)], axis=0)
        t = jnp.dot(ry, sp, precision=jax.lax.Precision.HIGHEST)
        pix_ref[b] = jnp.dot(t, rxt, precision=jax.lax.Precision.HIGHEST)


def _interp_matrices():
    yy = np.arange(H, dtype=np.float64) * (PH - 1) / (H - 1)
    y0 = np.floor(yy).astype(np.int64)
    fy = (yy - y0).astype(np.float32)
    y1 = np.minimum(y0 + 1, PH - 1)
    ry = np.zeros((H, PH), np.float32)
    ry[np.arange(H), y0] += 1.0 - fy
    ry[np.arange(H), y1] += fy
    return jnp.asarray(ry)


_OTHERS_NP = np.stack([np.concatenate([np.arange(b), np.arange(b + 1, B)])
                       for b in range(B)])


@jax.jit
def kernel(features, cls_tokens):
    mins, amins = pl.pallas_call(
        _stage_a_body,
        grid=(L,),
        in_specs=[pl.BlockSpec((1, B, P, D), lambda l: (l, 0, 0, 0))],
        out_specs=[pl.BlockSpec((1, NR, B, B * P), lambda l: (l, 0, 0, 0)),
                   pl.BlockSpec((1, NR, B, B * P), lambda l: (l, 0, 0, 0))],
        out_shape=[jax.ShapeDtypeStruct((L, NR, B, B * P), jnp.float32),
                   jax.ShapeDtypeStruct((L, NR, B, B * P), jnp.int32)],
        interpret=_INTERPRET,
    )(features)

    ry = _interp_matrices()
    finals, pix = pl.pallas_call(
        _stage_c_body,
        out_shape=[jax.ShapeDtypeStruct((1, B), jnp.float32),
                   jax.ShapeDtypeStruct((B, H, W), jnp.float32)],
        interpret=_INTERPRET,
    )(mins, cls_tokens, ry, ry.T)

    am5 = amins.reshape(L, NR, B, B, P)
    rows = [am5[:, :, _OTHERS_NP[b], b, :] for b in range(B)]
    min_indices = jnp.stack(rows, axis=0)
    return finals.reshape(B), pix, min_indices
